# baseline (speedup 1.0000x reference)
import functools

import jax
import jax.numpy as jnp
from jax import lax
from jax.experimental import pallas as pl
from jax.experimental.pallas import tpu as pltpu
from jax.experimental.pallas import tpu_sc as plsc

N_USER = 10000
N_ITEM = 40000
N = N_USER + N_ITEM
E = 800000
D = 64
B = 4096
N_LAYERS = 3

H = D // 2
NS = 16
EP = E // NS
C = 400
KB = 80
J = C // KB
G = KB // 16
NITER = EP // C
NCHUNK = NS * NITER
RKB = 128
RPT = B // NS
RJ = RPT // RKB


def _sc_body(ego2, adjr, valr, idxu, idxp, idxn, outu, outp, outn,
             acc, dst0, dst1, dst2, dst3, src_a, src_b, val_a, val_b,
             rows_a, rows_b, idx_v, idx2_v, sem_m, sem_g, sem_s):
  cid = lax.axis_index("c")
  sid = lax.axis_index("s")
  dst_ring = (dst0, dst1, dst2, dst3)
  src_pp = (src_a, src_b)
  val_pp = (val_a, val_b)
  rows_pp = (rows_a, rows_b)

  zero16 = jnp.zeros((16,), jnp.float32)

  def zbody(r, carry):
    rows_a[r, pl.ds(0, 16)] = zero16
    rows_a[r, pl.ds(16, 16)] = zero16
    return carry

  lax.fori_loop(0, C, zbody, 0)
  zleft = N // NS
  for k in range(zleft // C):
    pltpu.sync_copy(rows_a, acc.at[pl.ds(sid * zleft + k * C, C)])
  pltpu.sync_copy(rows_a.at[pl.ds(0, zleft % C)],
                  acc.at[pl.ds(sid * zleft + (zleft // C) * C, zleft % C)])
  plsc.subcore_barrier()

  def fetch_meta(it, r4, p):
    base = (sid * NITER + it) * C
    pltpu.async_copy(adjr.at[0, pl.ds(base, C)], dst_ring[r4], sem_m)
    pltpu.async_copy(adjr.at[1, pl.ds(base, C)], src_pp[p], sem_m)
    pltpu.async_copy(valr.at[pl.ds(base, C)], val_pp[p], sem_m)

  def wait_meta(r4, p):
    pltpu.make_async_copy(adjr.at[0, pl.ds(0, C)], dst_ring[r4], sem_m).wait()
    pltpu.make_async_copy(adjr.at[1, pl.ds(0, C)], src_pp[p], sem_m).wait()
    pltpu.make_async_copy(valr.at[pl.ds(0, C)], val_pp[p], sem_m).wait()

  def compute_src2(p):
    srcb = src_pp[p]
    cvec = jnp.full((16,), 0, jnp.int32) + cid

    def sbody(g, carry):
      s16 = srcb[pl.ds(g * 16, 16)]
      srcb[pl.ds(g * 16, 16)] = s16 + s16 + cvec
      return carry

    lax.fori_loop(0, C // 16, sbody, 0)

  def fire_gather(p):
    for j in range(J):
      pltpu.async_copy(
          ego2.at[src_pp[p].at[pl.ds(j * KB, KB)]],
          rows_pp[p].at[pl.ds(j * KB, KB)], sem_g)

  def wait_gather(p):
    for j in range(J):
      pltpu.make_async_copy(
          ego2.at[src_pp[p].at[pl.ds(j * KB, KB)]],
          rows_pp[p].at[pl.ds(j * KB, KB)], sem_g).wait()

  def scale(p):
    valb, rowsb = val_pp[p], rows_pp[p]

    def grp_body(g, carry):
      v16 = valb[pl.ds(g * 16, 16)]
      for i in range(16):
        r = g * 16 + i
        bc = v16.at[jnp.full((16,), i, jnp.int32)].get(
            mode="promise_in_bounds")
        rowsb[r, pl.ds(0, 16)] = rowsb[r, pl.ds(0, 16)] * bc
        rowsb[r, pl.ds(16, 16)] = rowsb[r, pl.ds(16, 16)] * bc
      return carry

    lax.fori_loop(0, C // 16, grp_body, 0)

  def fire_scatter(r4, p):
    for j in range(J):
      pltpu.async_copy(
          rows_pp[p].at[pl.ds(j * KB, KB)],
          acc.at[dst_ring[r4].at[pl.ds(j * KB, KB)]], sem_s, add=True)

  def wait_scatter():
    for j in range(J):
      pltpu.make_async_copy(
          ego2.at[pl.ds(0, KB)], acc.at[pl.ds(0, KB)], sem_s).wait()

  fetch_meta(0, 0, 0)
  fetch_meta(1, 1, 1)
  wait_meta(0, 0)
  compute_src2(0)
  fire_gather(0)

  def pipe_body(it4, carry):
    for quad in range(4):
      it = it4 * 4 + quad
      r4 = quad
      p = quad % 2
      @pl.when(it > 0)
      def _():
        wait_scatter()
      @pl.when(it + 1 < NITER)
      def _():
        wait_meta((r4 + 1) % 4, 1 - p)
        compute_src2(1 - p)
        fire_gather(1 - p)
      wait_gather(p)
      scale(p)
      fire_scatter(r4, p)
      @pl.when(it + 2 < NITER)
      def _():
        fetch_meta(it + 2, (r4 + 2) % 4, p)
    return carry

  lax.fori_loop(0, NITER // 4, pipe_body, 0)
  wait_scatter()
  wait_gather(0)
  scale(0)
  fire_scatter(0, 0)
  wait_scatter()
  plsc.subcore_barrier()

  def readout(idxr, outr):
    for jj in range(RJ):
      row = sid * RJ + jj
      pltpu.sync_copy(idxr.at[pl.ds(row, 1)], idx_v)
      cvec = jnp.full((16,), 0, jnp.int32) + cid

      def ibody(g, carry):
        s16 = idx_v[0, pl.ds(g * 16, 16)]
        idx2_v[0, pl.ds(g * 16, 16)] = s16 + s16 + cvec
        return carry

      lax.fori_loop(0, RKB // 16, ibody, 0)
      dg = pltpu.async_copy(ego2.at[idx2_v.at[0]],
                            rows_a.at[pl.ds(0, RKB)], sem_g)
      dsde = pltpu.async_copy(acc.at[idx_v.at[0]],
                              rows_a.at[pl.ds(RKB, RKB)], sem_s)
      dg.wait()
      dsde.wait()

      def cbody(r, carry):
        for lo in (0, 16):
          e = rows_a[r, pl.ds(lo, 16)]
          s = rows_a[RKB + r, pl.ds(lo, 16)]
          rows_a[r, pl.ds(lo, 16)] = e * 0.25 + s * 0.75
        return carry

      lax.fori_loop(0, RKB, cbody, 0)
      pltpu.sync_copy(
          rows_a.at[pl.ds(0, RKB)],
          outr.at[pl.ds(sid * RPT + jj * RKB, RKB), pl.ds(cid * H, H)])

  readout(idxu, outu)
  readout(idxp, outp)
  readout(idxn, outn)


_sc_call = functools.partial(
    pl.kernel,
    mesh=plsc.VectorSubcoreMesh(core_axis_name="c", subcore_axis_name="s"),
    compiler_params=pltpu.CompilerParams(
        use_tc_tiling_on_sc=False, needs_layout_passes=False),
    out_type=(
        jax.ShapeDtypeStruct((B, D), jnp.float32),
        jax.ShapeDtypeStruct((B, D), jnp.float32),
        jax.ShapeDtypeStruct((B, D), jnp.float32),
    ),
    scratch_types=[
        pltpu.VMEM_SHARED((N, H), jnp.float32),
        pltpu.VMEM((C,), jnp.int32),
        pltpu.VMEM((C,), jnp.int32),
        pltpu.VMEM((C,), jnp.int32),
        pltpu.VMEM((C,), jnp.int32),
        pltpu.VMEM((C,), jnp.int32),
        pltpu.VMEM((C,), jnp.int32),
        pltpu.VMEM((C,), jnp.float32),
        pltpu.VMEM((C,), jnp.float32),
        pltpu.VMEM((C, H), jnp.float32),
        pltpu.VMEM((C, H), jnp.float32),
        pltpu.VMEM((1, RKB), jnp.int32),
        pltpu.VMEM((1, RKB), jnp.int32),
        pltpu.SemaphoreType.DMA,
        pltpu.SemaphoreType.DMA,
        pltpu.SemaphoreType.DMA,
    ],
)(_sc_body)


def kernel(user_emb, item_emb, adj_indices, adj_values, users, pos_items,
           neg_items):
  ego2 = jnp.concatenate([user_emb.reshape(2 * N_USER, H),
                          item_emb.reshape(2 * N_ITEM, H)], axis=0)
  adjr = adj_indices.astype(jnp.int32)
  valr = adj_values.astype(jnp.float32)
  idxu = users.astype(jnp.int32).reshape(B // RKB, RKB)
  idxp = (pos_items.astype(jnp.int32) + N_USER).reshape(B // RKB, RKB)
  idxn = (neg_items.astype(jnp.int32) + N_USER).reshape(B // RKB, RKB)
  return _sc_call(ego2, adjr, valr, idxu, idxp, idxn)


# speedup vs baseline: 1.2798x; 1.2798x over previous
import functools

import jax
import jax.numpy as jnp
from jax import lax
from jax.experimental import pallas as pl
from jax.experimental.pallas import tpu as pltpu
from jax.experimental.pallas import tpu_sc as plsc

N_USER = 10000
N_ITEM = 40000
N = N_USER + N_ITEM
E = 800000
D = 64
B = 4096
N_LAYERS = 3

H = D // 2
NS = 16
EP = E // NS
C = 400
KB = 80
J = C // KB
G = KB // 16
NITER = EP // C
NCHUNK = NS * NITER
RKB = 128
RPT = B // NS
RJ = RPT // RKB


def _sc_body(ego2, adjr, valr, idxu, idxp, idxn, outu, outp, outn,
             acc, dst0, dst1, dst2, dst3, src_a, src_b, val_a, val_b,
             rows_a, rows_b, idx_v, idx2_v, sem_m, sem_g, sem_s):
  cid = lax.axis_index("c")
  sid = lax.axis_index("s")
  dst_ring = (dst0, dst1, dst2, dst3)
  src_pp = (src_a, src_b)
  val_pp = (val_a, val_b)
  rows_pp = (rows_a, rows_b)

  zero16 = jnp.zeros((16,), jnp.float32)

  def zbody(r, carry):
    rows_a[r, pl.ds(0, 16)] = zero16
    rows_a[r, pl.ds(16, 16)] = zero16
    return carry

  lax.fori_loop(0, C, zbody, 0)
  zleft = N // NS
  for k in range(zleft // C):
    pltpu.sync_copy(rows_a, acc.at[pl.ds(sid * zleft + k * C, C)])
  pltpu.sync_copy(rows_a.at[pl.ds(0, zleft % C)],
                  acc.at[pl.ds(sid * zleft + (zleft // C) * C, zleft % C)])
  plsc.subcore_barrier()

  def fetch_meta(it, r4, p):
    base = (sid * NITER + it) * C
    pltpu.async_copy(adjr.at[0, pl.ds(base, C)], dst_ring[r4], sem_m)
    pltpu.async_copy(adjr.at[1, pl.ds(base, C)], src_pp[p], sem_m)
    pltpu.async_copy(valr.at[pl.ds(base, C)], val_pp[p], sem_m)

  def wait_meta(r4, p):
    pltpu.make_async_copy(adjr.at[0, pl.ds(0, C)], dst_ring[r4], sem_m).wait()
    pltpu.make_async_copy(adjr.at[1, pl.ds(0, C)], src_pp[p], sem_m).wait()
    pltpu.make_async_copy(valr.at[pl.ds(0, C)], val_pp[p], sem_m).wait()

  def compute_src2(p):
    srcb = src_pp[p]
    cvec = jnp.full((16,), 0, jnp.int32) + cid

    def sbody(g, carry):
      s16 = srcb[pl.ds(g * 16, 16)]
      srcb[pl.ds(g * 16, 16)] = s16 + s16 + cvec
      return carry

    lax.fori_loop(0, C // 16, sbody, 0)

  def fire_gather(p):
    for j in range(J):
      pltpu.async_copy(
          ego2.at[src_pp[p].at[pl.ds(j * KB, KB)]],
          rows_pp[p].at[pl.ds(j * KB, KB)], sem_g)

  def wait_gather(p):
    for j in range(J):
      pltpu.make_async_copy(
          ego2.at[src_pp[p].at[pl.ds(j * KB, KB)]],
          rows_pp[p].at[pl.ds(j * KB, KB)], sem_g).wait()

  def scale(p):
    valb, rowsb = val_pp[p], rows_pp[p]

    def grp_body(g, carry):
      v16 = valb[pl.ds(g * 16, 16)]
      for i in range(16):
        r = g * 16 + i
        bc = v16.at[jnp.full((16,), i, jnp.int32)].get(
            mode="promise_in_bounds")
        rowsb[r, pl.ds(0, 16)] = rowsb[r, pl.ds(0, 16)] * bc
        rowsb[r, pl.ds(16, 16)] = rowsb[r, pl.ds(16, 16)] * bc
      return carry

    lax.fori_loop(0, C // 16, grp_body, 0)

  def fire_scatter(r4, p):
    for j in range(J):
      pltpu.async_copy(
          rows_pp[p].at[pl.ds(j * KB, KB)],
          acc.at[dst_ring[r4].at[pl.ds(j * KB, KB)]], sem_s, add=True)

  def wait_scatter():
    for j in range(J):
      pltpu.make_async_copy(
          ego2.at[pl.ds(0, KB)], acc.at[pl.ds(0, KB)], sem_s).wait()

  fetch_meta(0, 0, 0)
  fetch_meta(1, 1, 1)
  wait_meta(0, 0)
  compute_src2(0)
  fire_gather(0)

  def pipe_body(it4, carry):
    for quad in range(4):
      it = it4 * 4 + quad
      r4 = quad
      p = quad % 2
      @pl.when(it > 0)
      def _():
        wait_scatter()
      @pl.when(it + 1 < NITER)
      def _():
        wait_meta((r4 + 1) % 4, 1 - p)
        compute_src2(1 - p)
        fire_gather(1 - p)
      wait_gather(p)
      scale(p)
      fire_scatter(r4, p)
      @pl.when(it + 2 < NITER)
      def _():
        fetch_meta(it + 2, (r4 + 2) % 4, p)
    return carry

  lax.fori_loop(0, NITER // 4, pipe_body, 0)
  wait_scatter()
  wait_gather(0)
  scale(0)
  fire_scatter(0, 0)
  wait_scatter()
  plsc.subcore_barrier()

  def readout(idxr, outr):
    for jj in range(RJ):
      row = sid * RJ + jj
      pltpu.sync_copy(idxr.at[pl.ds(row, 1)], idx_v)
      cvec = jnp.full((16,), 0, jnp.int32) + cid

      def ibody(g, carry):
        s16 = idx_v[0, pl.ds(g * 16, 16)]
        idx2_v[0, pl.ds(g * 16, 16)] = s16 + s16 + cvec
        return carry

      lax.fori_loop(0, RKB // 16, ibody, 0)
      dg = pltpu.async_copy(ego2.at[idx2_v.at[0]],
                            rows_a.at[pl.ds(0, RKB)], sem_g)
      dsde = pltpu.async_copy(acc.at[idx_v.at[0]],
                              rows_a.at[pl.ds(RKB, RKB)], sem_s)
      dg.wait()
      dsde.wait()

      def cbody(r, carry):
        for lo in (0, 16):
          e = rows_a[r, pl.ds(lo, 16)]
          s = rows_a[RKB + r, pl.ds(lo, 16)]
          rows_a[r, pl.ds(lo, 16)] = e * 0.25 + s * 0.75
        return carry

      lax.fori_loop(0, RKB, cbody, 0)
      pltpu.sync_copy(
          rows_a.at[pl.ds(0, RKB)],
          outr.at[pl.ds(sid * RPT + jj * RKB, RKB), pl.ds(cid * H, H)])

  readout(idxu, outu)
  readout(idxp, outp)
  readout(idxn, outn)


_sc_call = functools.partial(
    pl.kernel,
    mesh=plsc.VectorSubcoreMesh(core_axis_name="c", subcore_axis_name="s"),
    compiler_params=pltpu.CompilerParams(
        use_tc_tiling_on_sc=False, needs_layout_passes=False),
    out_type=(
        jax.ShapeDtypeStruct((B, D), jnp.float32),
        jax.ShapeDtypeStruct((B, D), jnp.float32),
        jax.ShapeDtypeStruct((B, D), jnp.float32),
    ),
    scratch_types=[
        pltpu.VMEM_SHARED((N, H), jnp.float32),
        pltpu.VMEM((C,), jnp.int32),
        pltpu.VMEM((C,), jnp.int32),
        pltpu.VMEM((C,), jnp.int32),
        pltpu.VMEM((C,), jnp.int32),
        pltpu.VMEM((C,), jnp.int32),
        pltpu.VMEM((C,), jnp.int32),
        pltpu.VMEM((C,), jnp.float32),
        pltpu.VMEM((C,), jnp.float32),
        pltpu.VMEM((C, H), jnp.float32),
        pltpu.VMEM((C, H), jnp.float32),
        pltpu.VMEM((1, RKB), jnp.int32),
        pltpu.VMEM((1, RKB), jnp.int32),
        pltpu.SemaphoreType.DMA,
        pltpu.SemaphoreType.DMA,
        pltpu.SemaphoreType.DMA,
    ],
)(_sc_body)


def kernel(user_emb, item_emb, adj_indices, adj_values, users, pos_items,
           neg_items):
  ego2 = jnp.concatenate([user_emb, item_emb], axis=0).reshape(2 * N, H)
  adjr = adj_indices.astype(jnp.int32)
  valr = adj_values.astype(jnp.float32)
  idxu = users.astype(jnp.int32).reshape(B // RKB, RKB)
  idxp = (pos_items.astype(jnp.int32) + N_USER).reshape(B // RKB, RKB)
  idxn = (neg_items.astype(jnp.int32) + N_USER).reshape(B // RKB, RKB)
  return _sc_call(ego2, adjr, valr, idxu, idxp, idxn)
